# Initial kernel scaffold; baseline (speedup 1.0000x reference)
import jax
import jax.numpy as jnp
from jax.experimental import pallas as pl


def kernel(z, W):
    raise NotImplementedError("write your pallas kernel here")


# speedup vs baseline: 1.9464x; 1.9464x over previous
import functools

import jax
import jax.numpy as jnp
from jax.experimental import pallas as pl
from jax.experimental.pallas import tpu as pltpu

N_E = 1024
D = 64
B = 16
HW = 1024
BETA_ = 0.25


def _zsq_tree(zb):
    s = zb * zb
    for m in (32, 16, 8):
        s3 = s.reshape(m, 2, s.shape[-1])
        s = s3[:, 0, :] + s3[:, 1, :]
    acc = s[0:1]
    for g in range(1, 8):
        acc = acc + s[g:g + 1]
    return acc


def _vq_body(z_ref, w_ref, idx_ref, zq_ref, loss_ref):
    b = pl.program_id(0)
    zb = z_ref[0]
    zsq = _zsq_tree(zb)
    w = w_ref[...]
    wsq = jnp.sum(w * w, axis=1, keepdims=True)
    prod = jax.lax.dot_general(w, zb, (((1,), (0,)), ((), ())),
                               preferred_element_type=jnp.float32)
    d = (zsq + wsq) - 2.0 * prod
    mind = jnp.min(d, axis=0, keepdims=True)
    kiota = jax.lax.broadcasted_iota(jnp.int32, (N_E, HW), 0)
    idx = jnp.min(jnp.where(d == mind, kiota, N_E), axis=0)
    onehot = (jax.lax.broadcasted_iota(jnp.int32, (N_E, HW), 0)
              == idx[None, :]).astype(jnp.float32)
    zq = jax.lax.dot_general(w, onehot, (((0,), (0,)), ((), ())),
                             preferred_element_type=jnp.float32)
    zq_ref[0] = zb + (zq - zb)
    idx_ref[0, 0] = idx

    @pl.when(b == 0)
    def _():
        loss_ref[...] = jnp.zeros_like(loss_ref)

    loss_ref[...] += jnp.sum((zq - zb) ** 2).reshape(1, 1)


def kernel(z, W):
    z3 = z.reshape(B, D, HW)
    idx3, zq3, losssum = pl.pallas_call(
        _vq_body,
        grid=(B,),
        in_specs=[
            pl.BlockSpec((1, D, HW), lambda b: (b, 0, 0)),
            pl.BlockSpec((N_E, D), lambda b: (0, 0)),
        ],
        out_specs=[
            pl.BlockSpec((1, 1, HW), lambda b: (b, 0, 0)),
            pl.BlockSpec((1, D, HW), lambda b: (b, 0, 0)),
            pl.BlockSpec((1, 1), lambda b: (0, 0)),
        ],
        out_shape=[
            jax.ShapeDtypeStruct((B, 1, HW), jnp.int32),
            jax.ShapeDtypeStruct((B, D, HW), jnp.float32),
            jax.ShapeDtypeStruct((1, 1), jnp.float32),
        ],
    )(z3, W)
    loss = (1.0 + BETA_) * losssum[0, 0] / (B * D * HW)
    return (zq3.reshape(z.shape), loss, idx3.reshape(B * HW))
